# baseline (speedup 1.0000x reference)
import functools

import numpy as np

import jax
import jax.numpy as jnp
from jax import lax
from jax.experimental import pallas as pl
from jax.experimental.pallas import tpu as pltpu
from jax.experimental.pallas import tpu_sc as plsc

_VOCAB = 100000
_DIM = 64
_CTX = 20
_BATCH = 16384
_N_LABELS = 1024

_NC = 2
_NS = 16
_NW = _NC * _NS
_LANES = 16
_HALF = _BATCH // 2
_QCOLS = 2048
_NQ = _HALF // _QCOLS
_T = _CTX * _NQ
_GRPQ = _QCOLS // _LANES
_NBUF = 4
_HIMASK = np.uint32(0xFFFF0000)


def _make_sc_body(h, with_y):
    def _sc_body(*refs):
        if with_y:
            (inputs_hbm, labels_hbm, packx_hbm, wyt_hbm, xsum_hbm, y_hbm,
             row_v, idx_a, idx_b, idx_c, idx_d, out_a, out_b, lab_v, yrow_v,
             sem_r, sem_ia, sem_ib, sem_ic, sem_id) = refs
        else:
            (inputs_hbm, packx_hbm, xsum_hbm,
             row_v, idx_a, idx_b, idx_c, idx_d, out_a, out_b, lab_v, yrow_v,
             sem_r, sem_ia, sem_ib, sem_ic, sem_id) = refs
        wid = lax.axis_index("s") * _NC + lax.axis_index("c")

        idx_bufs = (idx_a, idx_b, idx_c, idx_d)
        idx_sems = (sem_ia, sem_ib, sem_ic, sem_id)

        if with_y:
            pltpu.sync_copy(labels_hbm, lab_v)

        pltpu.async_copy(packx_hbm.at[wid], row_v, sem_r).wait()

        zeros = jnp.zeros((_LANES,), jnp.float32)

        def start_idx(t, p):
            c = t // _NQ
            q = t - c * _NQ
            pltpu.async_copy(
                inputs_hbm.at[c, pl.ds(h * _HALF + q * _QCOLS, _QCOLS)],
                idx_bufs[p], idx_sems[p])

        def wait_idx(p):
            pltpu.make_async_copy(inputs_hbm.at[0, pl.ds(0, _QCOLS)],
                                  idx_bufs[p], idx_sems[p]).wait()

        def process(t, p):
            idx_v = idx_bufs[p]
            qbase = (t % _NQ) * _QCOLS

            @plsc.parallel_loop(0, _GRPQ, unroll=8)
            def _grp(g):
                sl = pl.ds(qbase + g * _LANES, _LANES)
                iv = idx_v[pl.ds(g * _LANES, _LANES)]
                vals = plsc.load_gather(row_v, [iv])
                u = plsc.bitcast(vals, jnp.uint32)
                va = plsc.bitcast(u & _HIMASK, jnp.float32)
                vb = plsc.bitcast(u << 16, jnp.float32)
                plsc.addupdate(out_a.at[sl], va)
                plsc.addupdate(out_b.at[sl], vb)

        @plsc.parallel_loop(0, _HALF // _LANES, unroll=4)
        def _zero(g):
            out_a[pl.ds(g * _LANES, _LANES)] = zeros
            out_b[pl.ds(g * _LANES, _LANES)] = zeros

        for p in range(_NBUF):
            start_idx(p, p)

        def quad(s, _):
            t = _NBUF * s
            for p in range(_NBUF):
                wait_idx(p)
                process(t + p, p)
                start_idx(t + p + _NBUF, p)
            return 0

        lax.fori_loop(0, _T // _NBUF - 1, quad, 0)

        t_last = _T - _NBUF
        for p in range(_NBUF):
            wait_idx(p)
            process(t_last + p, p)

        pltpu.sync_copy(out_a, xsum_hbm.at[wid])
        pltpu.sync_copy(out_b, xsum_hbm.at[wid + _NW])

        if with_y:
            for di in range(2):
                d = wid + di * _NW
                pltpu.async_copy(wyt_hbm.at[d], row_v, sem_r).wait()

                @plsc.parallel_loop(0, _N_LABELS // _LANES, unroll=2)
                def _lab(g):
                    sl = pl.ds(g * _LANES, _LANES)
                    vals = plsc.load_gather(row_v, [lab_v[sl]])
                    yrow_v[sl] = vals * (1.0 / _CTX)

                pltpu.sync_copy(yrow_v, y_hbm.at[d])

    return _sc_body


_SC_SCRATCH = [
    pltpu.VMEM((_VOCAB,), jnp.float32),
    pltpu.VMEM((_QCOLS,), jnp.int32),
    pltpu.VMEM((_QCOLS,), jnp.int32),
    pltpu.VMEM((_QCOLS,), jnp.int32),
    pltpu.VMEM((_QCOLS,), jnp.int32),
    pltpu.VMEM((_HALF,), jnp.float32),
    pltpu.VMEM((_HALF,), jnp.float32),
    pltpu.VMEM((_N_LABELS,), jnp.int32),
    pltpu.VMEM((_N_LABELS,), jnp.float32),
    pltpu.SemaphoreType.DMA,
    pltpu.SemaphoreType.DMA,
    pltpu.SemaphoreType.DMA,
    pltpu.SemaphoreType.DMA,
    pltpu.SemaphoreType.DMA,
]

_sc_half0 = pl.kernel(
    _make_sc_body(0, True),
    out_type=[
        jax.ShapeDtypeStruct((_DIM, _HALF), jnp.float32),
        jax.ShapeDtypeStruct((_DIM, _N_LABELS), jnp.float32),
    ],
    mesh=plsc.VectorSubcoreMesh(core_axis_name="c", subcore_axis_name="s"),
    compiler_params=pltpu.CompilerParams(needs_layout_passes=False),
    scratch_types=list(_SC_SCRATCH),
)

_sc_half1 = pl.kernel(
    _make_sc_body(1, False),
    out_type=[jax.ShapeDtypeStruct((_DIM, _HALF), jnp.float32)],
    mesh=plsc.VectorSubcoreMesh(core_axis_name="c", subcore_axis_name="s"),
    compiler_params=pltpu.CompilerParams(needs_layout_passes=False),
    scratch_types=list(_SC_SCRATCH),
)


_PCOLS = 4096


def _rne_hi16(x):
    u = lax.bitcast_convert_type(x, jnp.uint32)
    return (u + np.uint32(0x7FFF) + ((u >> 16) & np.uint32(1))) & _HIMASK


def _pack_body(x_ref, o_ref):
    a = x_ref[0:_DIM // 2, :]
    b = x_ref[_DIM // 2:_DIM, :]
    packed = _rne_hi16(a) | (_rne_hi16(b) >> 16)
    o_ref[...] = lax.bitcast_convert_type(packed, jnp.float32)


def _pack(wxt):
    return pl.pallas_call(
        _pack_body,
        grid=(pl.cdiv(_VOCAB, _PCOLS),),
        in_specs=[pl.BlockSpec((_DIM, _PCOLS), lambda i: (0, i))],
        out_specs=pl.BlockSpec((_DIM // 2, _PCOLS), lambda i: (0, i)),
        out_shape=jax.ShapeDtypeStruct((_DIM // 2, _VOCAB), jnp.float32),
    )(wxt)


_TC_BLOCK = 2048


def _tc_body(x_ref, y_ref, o_ref):
    s = lax.dot_general(
        x_ref[...], y_ref[...],
        dimension_numbers=(((0,), (0,)), ((), ())),
        preferred_element_type=jnp.float32,
    )
    o_ref[...] = 0.5 + 0.5 * jnp.tanh(0.5 * s)


def _tc_body_acc(x_ref, y_ref, p_ref, o_ref):
    del p_ref
    _tc_body(x_ref, y_ref, o_ref)


_NBLK = _HALF // _TC_BLOCK


def kernel(inputs, labels, W_x, W_y):
    wxt = W_x.T
    packx = _pack(wxt)
    x0, y_scaledT = _sc_half0(inputs, labels, packx, W_y.T)
    x1, = _sc_half1(inputs, packx)
    p0 = pl.pallas_call(
        _tc_body,
        grid=(_NBLK,),
        in_specs=[
            pl.BlockSpec((_DIM, _TC_BLOCK), lambda i: (0, i)),
            pl.BlockSpec((_DIM, _N_LABELS), lambda i: (0, 0)),
        ],
        out_specs=pl.BlockSpec((_TC_BLOCK, _N_LABELS), lambda i: (i, 0)),
        out_shape=jax.ShapeDtypeStruct((_BATCH, _N_LABELS), jnp.float32),
    )(x0, y_scaledT)
    prob = pl.pallas_call(
        _tc_body_acc,
        grid=(_NBLK,),
        in_specs=[
            pl.BlockSpec((_DIM, _TC_BLOCK), lambda i: (0, i)),
            pl.BlockSpec((_DIM, _N_LABELS), lambda i: (0, 0)),
            pl.BlockSpec(memory_space=pltpu.HBM),
        ],
        out_specs=pl.BlockSpec((_TC_BLOCK, _N_LABELS), lambda i: (i + _NBLK, 0)),
        out_shape=jax.ShapeDtypeStruct((_BATCH, _N_LABELS), jnp.float32),
        input_output_aliases={2: 0},
    )(x1, y_scaledT, p0)
    return prob


# speedup vs baseline: 1.0316x; 1.0316x over previous
import functools

import numpy as np

import jax
import jax.numpy as jnp
from jax import lax
from jax.experimental import pallas as pl
from jax.experimental.pallas import tpu as pltpu
from jax.experimental.pallas import tpu_sc as plsc

_VOCAB = 100000
_DIM = 64
_CTX = 20
_BATCH = 16384
_N_LABELS = 1024

_NC = 2
_NS = 16
_NW = _NC * _NS
_LANES = 16
_HALF = _BATCH // 2
_QCOLS = 2048
_NQ = _HALF // _QCOLS
_T = _CTX * _NQ
_GRPQ = _QCOLS // _LANES
_NBUF = 4
_HIMASK = np.uint32(0xFFFF0000)


def _make_sc_body(h, with_y):
    def _sc_body(*refs):
        if with_y:
            (inputs_hbm, labels_hbm, packx_hbm, wyt_hbm, xsum_hbm, y_hbm,
             row_v, idx_a, idx_b, idx_c, idx_d, out_a, out_b, lab_v, yrow_v,
             sem_r, sem_ia, sem_ib, sem_ic, sem_id) = refs
        else:
            (inputs_hbm, packx_hbm, xsum_hbm,
             row_v, idx_a, idx_b, idx_c, idx_d, out_a, out_b, lab_v, yrow_v,
             sem_r, sem_ia, sem_ib, sem_ic, sem_id) = refs
        wid = lax.axis_index("s") * _NC + lax.axis_index("c")

        idx_bufs = (idx_a, idx_b, idx_c, idx_d)
        idx_sems = (sem_ia, sem_ib, sem_ic, sem_id)

        if with_y:
            pltpu.sync_copy(labels_hbm, lab_v)

        pltpu.async_copy(packx_hbm.at[wid], row_v, sem_r).wait()

        zeros = jnp.zeros((_LANES,), jnp.float32)

        def start_idx(t, p):
            c = t // _NQ
            q = t - c * _NQ
            pltpu.async_copy(
                inputs_hbm.at[c, pl.ds(h * _HALF + q * _QCOLS, _QCOLS)],
                idx_bufs[p], idx_sems[p])

        def wait_idx(p):
            pltpu.make_async_copy(inputs_hbm.at[0, pl.ds(0, _QCOLS)],
                                  idx_bufs[p], idx_sems[p]).wait()

        def process(t, p):
            idx_v = idx_bufs[p]
            qbase = (t % _NQ) * _QCOLS

            @plsc.parallel_loop(0, _GRPQ, unroll=8)
            def _grp(g):
                sl = pl.ds(qbase + g * _LANES, _LANES)
                iv = idx_v[pl.ds(g * _LANES, _LANES)]
                vals = plsc.load_gather(row_v, [iv])
                u = plsc.bitcast(vals, jnp.uint32)
                va = plsc.bitcast(u & _HIMASK, jnp.float32)
                vb = plsc.bitcast(u << 16, jnp.float32)
                plsc.addupdate(out_a.at[sl], va)
                plsc.addupdate(out_b.at[sl], vb)

        @plsc.parallel_loop(0, _HALF // _LANES, unroll=4)
        def _zero(g):
            out_a[pl.ds(g * _LANES, _LANES)] = zeros
            out_b[pl.ds(g * _LANES, _LANES)] = zeros

        for p in range(_NBUF):
            start_idx(p, p)

        def quad(s, _):
            t = _NBUF * s
            for p in range(_NBUF):
                wait_idx(p)
                process(t + p, p)
                start_idx(t + p + _NBUF, p)
            return 0

        lax.fori_loop(0, _T // _NBUF - 1, quad, 0)

        t_last = _T - _NBUF
        for p in range(_NBUF):
            wait_idx(p)
            process(t_last + p, p)

        pltpu.sync_copy(out_a, xsum_hbm.at[wid])
        pltpu.sync_copy(out_b, xsum_hbm.at[wid + _NW])

        if with_y:
            for di in range(2):
                d = wid + di * _NW
                pltpu.async_copy(wyt_hbm.at[d], row_v, sem_r).wait()

                @plsc.parallel_loop(0, _N_LABELS // _LANES, unroll=2)
                def _lab(g):
                    sl = pl.ds(g * _LANES, _LANES)
                    vals = plsc.load_gather(row_v, [lab_v[sl]])
                    yrow_v[sl] = vals * (1.0 / _CTX)

                pltpu.sync_copy(yrow_v, y_hbm.at[d])

    return _sc_body


_SC_SCRATCH = [
    pltpu.VMEM((_VOCAB,), jnp.float32),
    pltpu.VMEM((_QCOLS,), jnp.int32),
    pltpu.VMEM((_QCOLS,), jnp.int32),
    pltpu.VMEM((_QCOLS,), jnp.int32),
    pltpu.VMEM((_QCOLS,), jnp.int32),
    pltpu.VMEM((_HALF,), jnp.float32),
    pltpu.VMEM((_HALF,), jnp.float32),
    pltpu.VMEM((_N_LABELS,), jnp.int32),
    pltpu.VMEM((_N_LABELS,), jnp.float32),
    pltpu.SemaphoreType.DMA,
    pltpu.SemaphoreType.DMA,
    pltpu.SemaphoreType.DMA,
    pltpu.SemaphoreType.DMA,
    pltpu.SemaphoreType.DMA,
]

_sc_half0 = pl.kernel(
    _make_sc_body(0, False),
    out_type=[jax.ShapeDtypeStruct((_DIM, _HALF), jnp.float32)],
    mesh=plsc.VectorSubcoreMesh(core_axis_name="c", subcore_axis_name="s"),
    compiler_params=pltpu.CompilerParams(needs_layout_passes=False),
    scratch_types=list(_SC_SCRATCH),
)


def _sc_y_body(labels_hbm, wyt_hbm, y_hbm, row_v, lab_v, yrow_v, sem_r):
    wid = lax.axis_index("s") * _NC + lax.axis_index("c")
    pltpu.sync_copy(labels_hbm, lab_v)
    for di in range(2):
        d = wid + di * _NW
        pltpu.async_copy(wyt_hbm.at[d], row_v, sem_r).wait()

        @plsc.parallel_loop(0, _N_LABELS // _LANES, unroll=2)
        def _lab(g):
            sl = pl.ds(g * _LANES, _LANES)
            vals = plsc.load_gather(row_v, [lab_v[sl]])
            yrow_v[sl] = vals * (1.0 / _CTX)

        pltpu.sync_copy(yrow_v, y_hbm.at[d])


_sc_y = pl.kernel(
    _sc_y_body,
    out_type=[jax.ShapeDtypeStruct((_DIM, _N_LABELS), jnp.float32)],
    mesh=plsc.VectorSubcoreMesh(core_axis_name="c", subcore_axis_name="s"),
    compiler_params=pltpu.CompilerParams(needs_layout_passes=False),
    scratch_types=[
        pltpu.VMEM((_VOCAB,), jnp.float32),
        pltpu.VMEM((_N_LABELS,), jnp.int32),
        pltpu.VMEM((_N_LABELS,), jnp.float32),
        pltpu.SemaphoreType.DMA,
    ],
)

_sc_half1 = pl.kernel(
    _make_sc_body(1, False),
    out_type=[jax.ShapeDtypeStruct((_DIM, _HALF), jnp.float32)],
    mesh=plsc.VectorSubcoreMesh(core_axis_name="c", subcore_axis_name="s"),
    compiler_params=pltpu.CompilerParams(needs_layout_passes=False),
    scratch_types=list(_SC_SCRATCH),
)


_PCOLS = 4096


def _rne_hi16(x):
    u = lax.bitcast_convert_type(x, jnp.uint32)
    return (u + np.uint32(0x7FFF) + ((u >> 16) & np.uint32(1))) & _HIMASK


def _pack_body(x_ref, o_ref):
    a = x_ref[0:_DIM // 2, :]
    b = x_ref[_DIM // 2:_DIM, :]
    packed = _rne_hi16(a) | (_rne_hi16(b) >> 16)
    o_ref[...] = lax.bitcast_convert_type(packed, jnp.float32)


def _pack(wxt):
    return pl.pallas_call(
        _pack_body,
        grid=(pl.cdiv(_VOCAB, _PCOLS),),
        in_specs=[pl.BlockSpec((_DIM, _PCOLS), lambda i: (0, i))],
        out_specs=pl.BlockSpec((_DIM // 2, _PCOLS), lambda i: (0, i)),
        out_shape=jax.ShapeDtypeStruct((_DIM // 2, _VOCAB), jnp.float32),
    )(wxt)


_TC_BLOCK = 2048


def _tc_body(x_ref, y_ref, o_ref):
    s = lax.dot_general(
        x_ref[...], y_ref[...],
        dimension_numbers=(((0,), (0,)), ((), ())),
        preferred_element_type=jnp.float32,
    )
    o_ref[...] = 0.5 + 0.5 * jnp.tanh(0.5 * s)


def _tc_body_acc(x_ref, y_ref, p_ref, o_ref):
    del p_ref
    _tc_body(x_ref, y_ref, o_ref)


_NBLK = _HALF // _TC_BLOCK


def kernel(inputs, labels, W_x, W_y):
    y_scaledT, = _sc_y(labels, W_y.T)
    packx = _pack(W_x.T)
    x0, = _sc_half0(inputs, packx)
    x1, = _sc_half1(inputs, packx)
    p0 = pl.pallas_call(
        _tc_body,
        grid=(_NBLK,),
        in_specs=[
            pl.BlockSpec((_DIM, _TC_BLOCK), lambda i: (0, i)),
            pl.BlockSpec((_DIM, _N_LABELS), lambda i: (0, 0)),
        ],
        out_specs=pl.BlockSpec((_TC_BLOCK, _N_LABELS), lambda i: (i, 0)),
        out_shape=jax.ShapeDtypeStruct((_BATCH, _N_LABELS), jnp.float32),
    )(x0, y_scaledT)
    prob = pl.pallas_call(
        _tc_body_acc,
        grid=(_NBLK,),
        in_specs=[
            pl.BlockSpec((_DIM, _TC_BLOCK), lambda i: (0, i)),
            pl.BlockSpec((_DIM, _N_LABELS), lambda i: (0, 0)),
            pl.BlockSpec(memory_space=pltpu.HBM),
        ],
        out_specs=pl.BlockSpec((_TC_BLOCK, _N_LABELS), lambda i: (i + _NBLK, 0)),
        out_shape=jax.ShapeDtypeStruct((_BATCH, _N_LABELS), jnp.float32),
        input_output_aliases={2: 0},
    )(x1, y_scaledT, p0)
    return prob


# speedup vs baseline: 1.1128x; 1.0787x over previous
import functools

import numpy as np

import jax
import jax.numpy as jnp
from jax import lax
from jax.experimental import pallas as pl
from jax.experimental.pallas import tpu as pltpu
from jax.experimental.pallas import tpu_sc as plsc

_VOCAB = 100000
_DIM = 64
_CTX = 20
_BATCH = 16384
_N_LABELS = 1024

_NC = 2
_NS = 16
_NW = _NC * _NS
_LANES = 16
_HALF = _BATCH // 2
_QCOLS = 2048
_NQ = _HALF // _QCOLS
_T = _CTX * _NQ
_GRPQ = _QCOLS // _LANES
_NBUF = 4
_HIMASK = np.uint32(0xFFFF0000)


def _make_sc_body(halves, with_y):
    if not isinstance(halves, tuple):
        halves = (halves,)

    def _sc_body(*refs):
        if with_y:
            (inputs_hbm, labels_hbm, packx_hbm, wyt_hbm, xsum_hbm, y_hbm,
             row_v, idx_a, idx_b, idx_c, idx_d, out_a, out_b, lab_v, yrow_v,
             sem_r, sem_ia, sem_ib, sem_ic, sem_id) = refs
        else:
            (inputs_hbm, packx_hbm, xsum_hbm,
             row_v, idx_a, idx_b, idx_c, idx_d, out_a, out_b, lab_v, yrow_v,
             sem_r, sem_ia, sem_ib, sem_ic, sem_id) = refs
        wid = lax.axis_index("s") * _NC + lax.axis_index("c")

        idx_bufs = (idx_a, idx_b, idx_c, idx_d)
        idx_sems = (sem_ia, sem_ib, sem_ic, sem_id)

        if with_y:
            pltpu.sync_copy(labels_hbm, lab_v)

        pltpu.async_copy(packx_hbm.at[wid], row_v, sem_r).wait()

        zeros = jnp.zeros((_LANES,), jnp.float32)

        def start_idx(h, t, p):
            c = t // _NQ
            q = t - c * _NQ
            pltpu.async_copy(
                inputs_hbm.at[c, pl.ds(h * _HALF + q * _QCOLS, _QCOLS)],
                idx_bufs[p], idx_sems[p])

        def wait_idx(p):
            pltpu.make_async_copy(inputs_hbm.at[0, pl.ds(0, _QCOLS)],
                                  idx_bufs[p], idx_sems[p]).wait()

        def process(t, p):
            idx_v = idx_bufs[p]
            qbase = (t % _NQ) * _QCOLS

            @plsc.parallel_loop(0, _GRPQ, unroll=8)
            def _grp(g):
                sl = pl.ds(qbase + g * _LANES, _LANES)
                iv = idx_v[pl.ds(g * _LANES, _LANES)]
                vals = plsc.load_gather(row_v, [iv])
                u = plsc.bitcast(vals, jnp.uint32)
                va = plsc.bitcast(u & _HIMASK, jnp.float32)
                vb = plsc.bitcast(u << 16, jnp.float32)
                plsc.addupdate(out_a.at[sl], va)
                plsc.addupdate(out_b.at[sl], vb)

        for h in halves:
            @plsc.parallel_loop(0, _HALF // _LANES, unroll=4)
            def _zero(g):
                out_a[pl.ds(g * _LANES, _LANES)] = zeros
                out_b[pl.ds(g * _LANES, _LANES)] = zeros

            for p in range(_NBUF):
                start_idx(h, p, p)

            def quad(s, _):
                t = _NBUF * s
                for p in range(_NBUF):
                    wait_idx(p)
                    process(t + p, p)
                    start_idx(h, t + p + _NBUF, p)
                return 0

            lax.fori_loop(0, _T // _NBUF - 1, quad, 0)

            t_last = _T - _NBUF
            for p in range(_NBUF):
                wait_idx(p)
                process(t_last + p, p)

            if len(halves) == 1:
                pltpu.sync_copy(out_a, xsum_hbm.at[wid])
                pltpu.sync_copy(out_b, xsum_hbm.at[wid + _NW])
            else:
                sl = pl.ds(h * _HALF, _HALF)
                pltpu.sync_copy(out_a, xsum_hbm.at[wid, sl])
                pltpu.sync_copy(out_b, xsum_hbm.at[wid + _NW, sl])

        if with_y:
            for di in range(2):
                d = wid + di * _NW
                pltpu.async_copy(wyt_hbm.at[d], row_v, sem_r).wait()

                @plsc.parallel_loop(0, _N_LABELS // _LANES, unroll=2)
                def _lab(g):
                    sl = pl.ds(g * _LANES, _LANES)
                    vals = plsc.load_gather(row_v, [lab_v[sl]])
                    yrow_v[sl] = vals * (1.0 / _CTX)

                pltpu.sync_copy(yrow_v, y_hbm.at[d])

    return _sc_body


_SC_SCRATCH = [
    pltpu.VMEM((_VOCAB,), jnp.float32),
    pltpu.VMEM((_QCOLS,), jnp.int32),
    pltpu.VMEM((_QCOLS,), jnp.int32),
    pltpu.VMEM((_QCOLS,), jnp.int32),
    pltpu.VMEM((_QCOLS,), jnp.int32),
    pltpu.VMEM((_HALF,), jnp.float32),
    pltpu.VMEM((_HALF,), jnp.float32),
    pltpu.VMEM((_N_LABELS,), jnp.int32),
    pltpu.VMEM((_N_LABELS,), jnp.float32),
    pltpu.SemaphoreType.DMA,
    pltpu.SemaphoreType.DMA,
    pltpu.SemaphoreType.DMA,
    pltpu.SemaphoreType.DMA,
    pltpu.SemaphoreType.DMA,
]

_sc_x_full = pl.kernel(
    _make_sc_body((0, 1), False),
    out_type=[jax.ShapeDtypeStruct((_DIM, _BATCH), jnp.float32)],
    mesh=plsc.VectorSubcoreMesh(core_axis_name="c", subcore_axis_name="s"),
    compiler_params=pltpu.CompilerParams(needs_layout_passes=False),
    scratch_types=list(_SC_SCRATCH),
)


def _sc_y_body(labels_hbm, wyt_hbm, y_hbm, row_v, lab_v, yrow_v, sem_r):
    wid = lax.axis_index("s") * _NC + lax.axis_index("c")
    pltpu.sync_copy(labels_hbm, lab_v)
    for di in range(2):
        d = wid + di * _NW
        pltpu.async_copy(wyt_hbm.at[d], row_v, sem_r).wait()

        @plsc.parallel_loop(0, _N_LABELS // _LANES, unroll=2)
        def _lab(g):
            sl = pl.ds(g * _LANES, _LANES)
            vals = plsc.load_gather(row_v, [lab_v[sl]])
            yrow_v[sl] = vals * (1.0 / _CTX)

        pltpu.sync_copy(yrow_v, y_hbm.at[d])


_sc_y = pl.kernel(
    _sc_y_body,
    out_type=[jax.ShapeDtypeStruct((_DIM, _N_LABELS), jnp.float32)],
    mesh=plsc.VectorSubcoreMesh(core_axis_name="c", subcore_axis_name="s"),
    compiler_params=pltpu.CompilerParams(needs_layout_passes=False),
    scratch_types=[
        pltpu.VMEM((_VOCAB,), jnp.float32),
        pltpu.VMEM((_N_LABELS,), jnp.int32),
        pltpu.VMEM((_N_LABELS,), jnp.float32),
        pltpu.SemaphoreType.DMA,
    ],
)

_PCOLS = 4096


def _rne_hi16(x):
    u = lax.bitcast_convert_type(x, jnp.uint32)
    return (u + np.uint32(0x7FFF) + ((u >> 16) & np.uint32(1))) & _HIMASK


def _pack_body(x_ref, o_ref):
    a = x_ref[0:_DIM // 2, :]
    b = x_ref[_DIM // 2:_DIM, :]
    packed = _rne_hi16(a) | (_rne_hi16(b) >> 16)
    o_ref[...] = lax.bitcast_convert_type(packed, jnp.float32)


def _pack(wxt):
    return pl.pallas_call(
        _pack_body,
        grid=(pl.cdiv(_VOCAB, _PCOLS),),
        in_specs=[pl.BlockSpec((_DIM, _PCOLS), lambda i: (0, i))],
        out_specs=pl.BlockSpec((_DIM // 2, _PCOLS), lambda i: (0, i)),
        out_shape=jax.ShapeDtypeStruct((_DIM // 2, _VOCAB), jnp.float32),
    )(wxt)


_TC_BLOCK = 2048


def _tc_body(x_ref, y_ref, o_ref):
    s = lax.dot_general(
        x_ref[...], y_ref[...],
        dimension_numbers=(((0,), (0,)), ((), ())),
        preferred_element_type=jnp.float32,
    )
    o_ref[...] = 0.5 + 0.5 * jnp.tanh(0.5 * s)


def _tc_body_acc(x_ref, y_ref, p_ref, o_ref):
    del p_ref
    _tc_body(x_ref, y_ref, o_ref)


_NBLK = _HALF // _TC_BLOCK


def kernel(inputs, labels, W_x, W_y):
    y_scaledT, = _sc_y(labels, W_y.T)
    packx = _pack(W_x.T)
    xsumT, = _sc_x_full(inputs, packx)
    prob = pl.pallas_call(
        _tc_body,
        grid=(_BATCH // _TC_BLOCK,),
        in_specs=[
            pl.BlockSpec((_DIM, _TC_BLOCK), lambda i: (0, i)),
            pl.BlockSpec((_DIM, _N_LABELS), lambda i: (0, 0)),
        ],
        out_specs=pl.BlockSpec((_TC_BLOCK, _N_LABELS), lambda i: (i, 0)),
        out_shape=jax.ShapeDtypeStruct((_BATCH, _N_LABELS), jnp.float32),
    )(xsumT, y_scaledT)
    return prob


# speedup vs baseline: 1.1400x; 1.0245x over previous
import functools

import numpy as np

import jax
import jax.numpy as jnp
from jax import lax
from jax.experimental import pallas as pl
from jax.experimental.pallas import tpu as pltpu
from jax.experimental.pallas import tpu_sc as plsc

_VOCAB = 100000
_DIM = 64
_CTX = 20
_BATCH = 16384
_N_LABELS = 1024

_NC = 2
_NS = 16
_NW = _NC * _NS
_LANES = 16
_HALF = _BATCH // 2
_QCOLS = 2048
_NQ = _HALF // _QCOLS
_T = _CTX * _NQ
_GRPQ = _QCOLS // _LANES
_NBUF = 6
_HIMASK = np.uint32(0xFFFF0000)


def _make_sc_body(halves, with_y):
    del with_y
    if not isinstance(halves, tuple):
        halves = (halves,)

    def _sc_body(*refs):
        (inputs_hbm, packx_hbm, xsum_hbm, row_v) = refs[:4]
        idx_bufs = refs[4:4 + _NBUF]
        out_a, out_b = refs[4 + _NBUF:6 + _NBUF]
        sem_r = refs[6 + _NBUF]
        idx_sems = refs[7 + _NBUF:7 + 2 * _NBUF]
        wid = lax.axis_index("s") * _NC + lax.axis_index("c")

        pltpu.async_copy(packx_hbm.at[wid], row_v, sem_r).wait()

        zeros = jnp.zeros((_LANES,), jnp.float32)

        def start_idx(h, t, p):
            c = t // _NQ
            q = t - c * _NQ
            pltpu.async_copy(
                inputs_hbm.at[c, pl.ds(h * _HALF + q * _QCOLS, _QCOLS)],
                idx_bufs[p], idx_sems[p])

        def wait_idx(p):
            pltpu.make_async_copy(inputs_hbm.at[0, pl.ds(0, _QCOLS)],
                                  idx_bufs[p], idx_sems[p]).wait()

        def process(t, p):
            idx_v = idx_bufs[p]
            qbase = (t % _NQ) * _QCOLS

            @plsc.parallel_loop(0, _GRPQ, unroll=8)
            def _grp(g):
                sl = pl.ds(qbase + g * _LANES, _LANES)
                iv = idx_v[pl.ds(g * _LANES, _LANES)]
                vals = plsc.load_gather(row_v, [iv])
                u = plsc.bitcast(vals, jnp.uint32)
                va = plsc.bitcast(u & _HIMASK, jnp.float32)
                vb = plsc.bitcast(u << 16, jnp.float32)
                plsc.addupdate(out_a.at[sl], va)
                plsc.addupdate(out_b.at[sl], vb)

        for h in halves:
            @plsc.parallel_loop(0, _HALF // _LANES, unroll=4)
            def _zero(g):
                out_a[pl.ds(g * _LANES, _LANES)] = zeros
                out_b[pl.ds(g * _LANES, _LANES)] = zeros

            for p in range(_NBUF):
                start_idx(h, p, p)

            def quad(s, _):
                t = _NBUF * s
                for p in range(_NBUF):
                    wait_idx(p)
                    process(t + p, p)
                    start_idx(h, t + p + _NBUF, p)
                return 0

            lax.fori_loop(0, _T // _NBUF - 1, quad, 0)

            t_last = _T - _NBUF
            for p in range(_NBUF):
                wait_idx(p)
                process(t_last + p, p)

            if len(halves) == 1:
                pltpu.sync_copy(out_a, xsum_hbm.at[wid])
                pltpu.sync_copy(out_b, xsum_hbm.at[wid + _NW])
            else:
                sl = pl.ds(h * _HALF, _HALF)
                pltpu.sync_copy(out_a, xsum_hbm.at[wid, sl])
                pltpu.sync_copy(out_b, xsum_hbm.at[wid + _NW, sl])

    return _sc_body


_SC_SCRATCH = (
    [pltpu.VMEM((_VOCAB,), jnp.float32)]
    + [pltpu.VMEM((_QCOLS,), jnp.int32)] * _NBUF
    + [pltpu.VMEM((_HALF,), jnp.float32)] * 2
    + [pltpu.SemaphoreType.DMA] * (1 + _NBUF)
)

_sc_x_full = pl.kernel(
    _make_sc_body((0, 1), False),
    out_type=[jax.ShapeDtypeStruct((_DIM, _BATCH), jnp.float32)],
    mesh=plsc.VectorSubcoreMesh(core_axis_name="c", subcore_axis_name="s"),
    compiler_params=pltpu.CompilerParams(needs_layout_passes=False),
    scratch_types=list(_SC_SCRATCH),
)


def _sc_y_body(labels_hbm, wyt_hbm, y_hbm, row_v, lab_v, yrow_v, sem_r):
    wid = lax.axis_index("s") * _NC + lax.axis_index("c")
    pltpu.sync_copy(labels_hbm, lab_v)
    for di in range(2):
        d = wid + di * _NW
        pltpu.async_copy(wyt_hbm.at[d], row_v, sem_r).wait()

        @plsc.parallel_loop(0, _N_LABELS // _LANES, unroll=2)
        def _lab(g):
            sl = pl.ds(g * _LANES, _LANES)
            vals = plsc.load_gather(row_v, [lab_v[sl]])
            yrow_v[sl] = vals * (1.0 / _CTX)

        pltpu.sync_copy(yrow_v, y_hbm.at[d])


_sc_y = pl.kernel(
    _sc_y_body,
    out_type=[jax.ShapeDtypeStruct((_DIM, _N_LABELS), jnp.float32)],
    mesh=plsc.VectorSubcoreMesh(core_axis_name="c", subcore_axis_name="s"),
    compiler_params=pltpu.CompilerParams(needs_layout_passes=False),
    scratch_types=[
        pltpu.VMEM((_VOCAB,), jnp.float32),
        pltpu.VMEM((_N_LABELS,), jnp.int32),
        pltpu.VMEM((_N_LABELS,), jnp.float32),
        pltpu.SemaphoreType.DMA,
    ],
)

_PCOLS = 4096


def _rne_hi16(x):
    u = lax.bitcast_convert_type(x, jnp.uint32)
    return (u + np.uint32(0x7FFF) + ((u >> 16) & np.uint32(1))) & _HIMASK


def _pack_body(x_ref, o_ref):
    a = x_ref[0:_DIM // 2, :]
    b = x_ref[_DIM // 2:_DIM, :]
    packed = _rne_hi16(a) | (_rne_hi16(b) >> 16)
    o_ref[...] = lax.bitcast_convert_type(packed, jnp.float32)


def _pack(wxt):
    return pl.pallas_call(
        _pack_body,
        grid=(pl.cdiv(_VOCAB, _PCOLS),),
        in_specs=[pl.BlockSpec((_DIM, _PCOLS), lambda i: (0, i))],
        out_specs=pl.BlockSpec((_DIM // 2, _PCOLS), lambda i: (0, i)),
        out_shape=jax.ShapeDtypeStruct((_DIM // 2, _VOCAB), jnp.float32),
    )(wxt)


_TC_BLOCK = 2048


def _tc_body(x_ref, y_ref, o_ref):
    s = lax.dot_general(
        x_ref[...], y_ref[...],
        dimension_numbers=(((0,), (0,)), ((), ())),
        preferred_element_type=jnp.float32,
    )
    o_ref[...] = 0.5 + 0.5 * jnp.tanh(0.5 * s)


def _tc_body_acc(x_ref, y_ref, p_ref, o_ref):
    del p_ref
    _tc_body(x_ref, y_ref, o_ref)


_NBLK = _HALF // _TC_BLOCK


def kernel(inputs, labels, W_x, W_y):
    y_scaledT, = _sc_y(labels, W_y.T)
    packx = _pack(W_x.T)
    xsumT, = _sc_x_full(inputs, packx)
    prob = pl.pallas_call(
        _tc_body,
        grid=(_BATCH // _TC_BLOCK,),
        in_specs=[
            pl.BlockSpec((_DIM, _TC_BLOCK), lambda i: (0, i)),
            pl.BlockSpec((_DIM, _N_LABELS), lambda i: (0, 0)),
        ],
        out_specs=pl.BlockSpec((_TC_BLOCK, _N_LABELS), lambda i: (i, 0)),
        out_shape=jax.ShapeDtypeStruct((_BATCH, _N_LABELS), jnp.float32),
    )(xsumT, y_scaledT)
    return prob
